# Initial kernel scaffold; baseline (speedup 1.0000x reference)
import jax
import jax.numpy as jnp
from jax.experimental import pallas as pl


def kernel(x, q_w, q_b, k_w, k_b, phi1_w, phi1_b, phi2_w, phi2_b, xi1_w, xi1_b, xi2_w, xi2_b, rho1_w, rho1_b, rho2_w, rho2_b):
    raise NotImplementedError("write your pallas kernel here")


# speedup vs baseline: 9.8643x; 9.8643x over previous
import functools

import jax
import jax.numpy as jnp
from jax.experimental import pallas as pl
from jax.experimental.pallas import tpu as pltpu

L = 2048
D = 64
TOPK = 64
TM = 256
NT = L // TM
RSUB = L // 128
NEG = float("-inf")


def _dot_t(a, w):
    return jax.lax.dot_general(a, w, (((1,), (1,)), ((), ())),
                               preferred_element_type=jnp.float32)


def _dot(a, b):
    return jax.lax.dot_general(a, b, (((1,), (0,)), ((), ())),
                               preferred_element_type=jnp.float32)


def _body(x_ref, qw_ref, qb_ref, kw_ref, kb_ref,
          p1w_ref, p1b_ref, p2w_ref, p2b_ref,
          x1w_ref, x1b_ref, x2w_ref, x2b_ref,
          r1w_ref, r1b_ref, r2w_ref, r2b_ref,
          out_ref,
          scores_ref, q_ref, k_ref, rowmax_ref, rowarg_ref,
          vals_ref, rows_ref, cols_ref):
    xb = x_ref[0]
    scale = jnp.float32(0.125)

    maskf = (jnp.sum(jnp.abs(xb), axis=1) != 0.0)
    q_ref[...] = _dot_t(xb, qw_ref[...]) + qb_ref[...]
    k_ref[...] = _dot_t(xb, kw_ref[...]) + kb_ref[...]

    ci = jax.lax.broadcasted_iota(jnp.int32, (TM, L), 1)

    for ti in range(NT):
        qt = q_ref[ti * TM:(ti + 1) * TM, :]
        s = _dot_t(qt, k_ref[...]) * scale
        m2 = maskf[ti * TM:(ti + 1) * TM][:, None] & maskf[None, :]
        s = jnp.where(m2, s, NEG)
        scores_ref[ti * TM:(ti + 1) * TM, :] = s
        rm = jnp.max(s, axis=1)
        pos = jnp.where(s == rm[:, None], ci, L)
        ra = jnp.min(pos, axis=1).astype(jnp.int32)
        r0 = ti * TM // 128
        rowmax_ref[r0:r0 + TM // 128, :] = rm.reshape(TM // 128, 128)
        rowarg_ref[r0:r0 + TM // 128, :] = ra.reshape(TM // 128, 128)

    flat2d = (jax.lax.broadcasted_iota(jnp.int32, (RSUB, 128), 0) * 128
              + jax.lax.broadcasted_iota(jnp.int32, (RSUB, 128), 1))
    ci_row = jax.lax.broadcasted_iota(jnp.int32, (1, L), 1)
    it64 = jax.lax.broadcasted_iota(jnp.int32, (1, TOPK), 1)

    def step(t, carry):
        rm = rowmax_ref[...]
        m = jnp.max(rm)
        pos = jnp.where(rm == m, flat2d, L * L)
        r = jnp.min(pos)
        sel = pos == r
        c = jnp.sum(jnp.where(sel, rowarg_ref[...], 0))
        vals_ref[...] = jnp.where(it64 == t, m, vals_ref[...])
        rows_ref[...] = jnp.where(it64 == t, r, rows_ref[...])
        cols_ref[...] = jnp.where(it64 == t, c, cols_ref[...])
        row = scores_ref[pl.ds(r, 1), :]
        row = jnp.where(ci_row == c, NEG, row)
        scores_ref[pl.ds(r, 1), :] = row
        nm = jnp.max(row)
        na = jnp.min(jnp.where(row == nm, ci_row, L)).astype(jnp.int32)
        rowmax_ref[...] = jnp.where(sel, nm, rm)
        rowarg_ref[...] = jnp.where(sel, na, rowarg_ref[...])
        return carry

    jax.lax.fori_loop(0, TOPK, step, 0)

    vals = vals_ref[...]
    rows = rows_ref[...]
    cols = cols_ref[...]
    e = jnp.exp(vals - jnp.max(vals))
    w = (e / jnp.sum(e)).reshape(TOPK, 1)

    io_l = jax.lax.broadcasted_iota(jnp.int32, (TOPK, L), 1)
    oh_i = (rows.reshape(TOPK, 1) == io_l).astype(jnp.float32)
    oh_j = (cols.reshape(TOPK, 1) == io_l).astype(jnp.float32)
    x_i = _dot(oh_i, xb)
    x_j = _dot(oh_j, xb)

    h_s = jax.nn.relu(_dot_t(x_i, p1w_ref[...]) + p1b_ref[...])
    f_s = _dot_t(h_s, p2w_ref[...]) + p2b_ref[...]

    x1w = x1w_ref[...]
    h_p = jax.nn.relu(_dot_t(x_i, x1w[:, :D]) + _dot_t(x_j, x1w[:, D:])
                      + x1b_ref[...])
    f_p = _dot_t(h_p, x2w_ref[...]) + x2b_ref[...]

    self_m = rows.reshape(TOPK, 1) == cols.reshape(TOPK, 1)
    inter = jnp.where(self_m, f_s, f_p) * w
    pooled = jnp.sum(inter, axis=0).reshape(1, D)

    o1 = jax.nn.relu(_dot_t(pooled, r1w_ref[...]) + r1b_ref[...])
    out_ref[0] = _dot_t(o1, r2w_ref[...]) + r2b_ref[...]


@jax.jit
def kernel(x, q_w, q_b, k_w, k_b, phi1_w, phi1_b, phi2_w, phi2_b,
           xi1_w, xi1_b, xi2_w, xi2_b, rho1_w, rho1_b, rho2_w, rho2_b):
    B = x.shape[0]
    b2 = lambda v: v.reshape(1, -1)
    w_spec = lambda a: pl.BlockSpec(a.shape, lambda b: (0,) * a.ndim)
    args = (x, q_w, b2(q_b), k_w, b2(k_b),
            phi1_w, b2(phi1_b), phi2_w, b2(phi2_b),
            xi1_w, b2(xi1_b), xi2_w, b2(xi2_b),
            rho1_w, b2(rho1_b), rho2_w, b2(rho2_b))
    in_specs = [pl.BlockSpec((1, L, D), lambda b: (b, 0, 0))]
    in_specs += [w_spec(a) for a in args[1:]]
    out = pl.pallas_call(
        _body,
        grid=(B,),
        in_specs=in_specs,
        out_specs=pl.BlockSpec((1, 1, D), lambda b: (b, 0, 0)),
        out_shape=jax.ShapeDtypeStruct((B, 1, D), jnp.float32),
        scratch_shapes=[
            pltpu.VMEM((L, L), jnp.float32),
            pltpu.VMEM((L, D), jnp.float32),
            pltpu.VMEM((L, D), jnp.float32),
            pltpu.VMEM((RSUB, 128), jnp.float32),
            pltpu.VMEM((RSUB, 128), jnp.int32),
            pltpu.VMEM((1, TOPK), jnp.float32),
            pltpu.VMEM((1, TOPK), jnp.int32),
            pltpu.VMEM((1, TOPK), jnp.int32),
        ],
        compiler_params=pltpu.CompilerParams(
            dimension_semantics=("arbitrary",)),
    )(*args)
    return out.reshape(B, D)


# speedup vs baseline: 9.8655x; 1.0001x over previous
import functools

import jax
import jax.numpy as jnp
from jax.experimental import pallas as pl
from jax.experimental.pallas import tpu as pltpu

L = 2048
D = 64
TOPK = 64
TM = 256
NT = L // TM
RSUB = L // 128
NEG = float("-inf")


def _dot_t(a, w):
    return jax.lax.dot_general(a, w, (((1,), (1,)), ((), ())),
                               preferred_element_type=jnp.float32)


def _dot(a, b):
    return jax.lax.dot_general(a, b, (((1,), (0,)), ((), ())),
                               preferred_element_type=jnp.float32)


def _body(x_ref, qw_ref, qb_ref, kw_ref, kb_ref,
          p1w_ref, p1b_ref, p2w_ref, p2b_ref,
          x1w_ref, x1b_ref, x2w_ref, x2b_ref,
          r1w_ref, r1b_ref, r2w_ref, r2b_ref,
          out_ref,
          scores_ref, q_ref, k_ref, rowmax_ref, rowarg_ref,
          vals_ref, rows_ref, cols_ref):
    xb = x_ref[0]
    scale = jnp.float32(0.125)

    maskf = (jnp.sum(jnp.abs(xb), axis=1) != 0.0)
    q_ref[...] = _dot_t(xb, qw_ref[...]) + qb_ref[...]
    k_ref[...] = _dot_t(xb, kw_ref[...]) + kb_ref[...]

    ci = jax.lax.broadcasted_iota(jnp.int32, (TM, L), 1)

    for ti in range(NT):
        qt = q_ref[ti * TM:(ti + 1) * TM, :]
        s = _dot_t(qt, k_ref[...]) * scale
        m2 = maskf[ti * TM:(ti + 1) * TM][:, None] & maskf[None, :]
        s = jnp.where(m2, s, NEG)
        scores_ref[ti * TM:(ti + 1) * TM, :] = s
        rm = jnp.max(s, axis=1)
        pos = jnp.where(s == rm[:, None], ci, L)
        ra = jnp.min(pos, axis=1).astype(jnp.int32)
        r0 = ti * TM // 128
        rowmax_ref[r0:r0 + TM // 128, :] = rm.reshape(TM // 128, 128)
        rowarg_ref[r0:r0 + TM // 128, :] = ra.reshape(TM // 128, 128)

    flat2d = (jax.lax.broadcasted_iota(jnp.int32, (RSUB, 128), 0) * 128
              + jax.lax.broadcasted_iota(jnp.int32, (RSUB, 128), 1))
    ci_row = jax.lax.broadcasted_iota(jnp.int32, (1, L), 1)
    it64 = jax.lax.broadcasted_iota(jnp.int32, (1, TOPK), 1)

    def step(t, carry):
        rm = rowmax_ref[...]
        m = jnp.max(rm)
        pos = jnp.where(rm == m, flat2d, L * L)
        r = jnp.min(pos)
        sel = pos == r
        c = jnp.sum(jnp.where(sel, rowarg_ref[...], 0))
        vals_ref[...] = jnp.where(it64 == t, m, vals_ref[...])
        rows_ref[...] = jnp.where(it64 == t, r, rows_ref[...])
        cols_ref[...] = jnp.where(it64 == t, c, cols_ref[...])
        row = scores_ref[pl.ds(r, 1), :]
        row = jnp.where(ci_row == c, NEG, row)
        scores_ref[pl.ds(r, 1), :] = row
        nm = jnp.max(row)
        na = jnp.min(jnp.where(row == nm, ci_row, L)).astype(jnp.int32)
        rowmax_ref[...] = jnp.where(sel, nm, rm)
        rowarg_ref[...] = jnp.where(sel, na, rowarg_ref[...])
        return carry

    jax.lax.fori_loop(0, TOPK, step, 0)

    vals = vals_ref[...]
    rows = rows_ref[...]
    cols = cols_ref[...]
    e = jnp.exp(vals - jnp.max(vals))
    w = (e / jnp.sum(e)).reshape(TOPK, 1)

    io_l = jax.lax.broadcasted_iota(jnp.int32, (TOPK, L), 1)
    oh_i = (rows.reshape(TOPK, 1) == io_l).astype(jnp.float32)
    oh_j = (cols.reshape(TOPK, 1) == io_l).astype(jnp.float32)
    x_i = _dot(oh_i, xb)
    x_j = _dot(oh_j, xb)

    h_s = jax.nn.relu(_dot_t(x_i, p1w_ref[...]) + p1b_ref[...])
    f_s = _dot_t(h_s, p2w_ref[...]) + p2b_ref[...]

    x1w = x1w_ref[...]
    h_p = jax.nn.relu(_dot_t(x_i, x1w[:, :D]) + _dot_t(x_j, x1w[:, D:])
                      + x1b_ref[...])
    f_p = _dot_t(h_p, x2w_ref[...]) + x2b_ref[...]

    self_m = rows.reshape(TOPK, 1) == cols.reshape(TOPK, 1)
    inter = jnp.where(self_m, f_s, f_p) * w
    pooled = jnp.sum(inter, axis=0).reshape(1, D)

    o1 = jax.nn.relu(_dot_t(pooled, r1w_ref[...]) + r1b_ref[...])
    out_ref[0] = _dot_t(o1, r2w_ref[...]) + r2b_ref[...]


@jax.jit
def kernel(x, q_w, q_b, k_w, k_b, phi1_w, phi1_b, phi2_w, phi2_b,
           xi1_w, xi1_b, xi2_w, xi2_b, rho1_w, rho1_b, rho2_w, rho2_b):
    B = x.shape[0]
    b2 = lambda v: v.reshape(1, -1)
    w_spec = lambda a: pl.BlockSpec(a.shape, lambda b: (0,) * a.ndim)
    args = (x, q_w, b2(q_b), k_w, b2(k_b),
            phi1_w, b2(phi1_b), phi2_w, b2(phi2_b),
            xi1_w, b2(xi1_b), xi2_w, b2(xi2_b),
            rho1_w, b2(rho1_b), rho2_w, b2(rho2_b))
    in_specs = [pl.BlockSpec((1, L, D), lambda b: (b, 0, 0))]
    in_specs += [w_spec(a) for a in args[1:]]
    out = pl.pallas_call(
        _body,
        grid=(B,),
        in_specs=in_specs,
        out_specs=pl.BlockSpec((1, 1, D), lambda b: (b, 0, 0)),
        out_shape=jax.ShapeDtypeStruct((B, 1, D), jnp.float32),
        scratch_shapes=[
            pltpu.VMEM((L, L), jnp.float32),
            pltpu.VMEM((L, D), jnp.float32),
            pltpu.VMEM((L, D), jnp.float32),
            pltpu.VMEM((RSUB, 128), jnp.float32),
            pltpu.VMEM((RSUB, 128), jnp.int32),
            pltpu.VMEM((1, TOPK), jnp.float32),
            pltpu.VMEM((1, TOPK), jnp.int32),
            pltpu.VMEM((1, TOPK), jnp.int32),
        ],
        compiler_params=pltpu.CompilerParams(
            dimension_semantics=("parallel",)),
    )(*args)
    return out.reshape(B, D)
